# baseline (speedup 1.0000x reference)
import functools

import jax
import jax.numpy as jnp
from jax import lax
from jax.experimental import pallas as pl
from jax.experimental.pallas import tpu as pltpu
from jax.experimental.pallas import tpu_sc as plsc

VOCAB = 100000
DIM = 128
B, L = 64, 16
N_TOK = B * L

V_BLK = 1024
N_VBLK = -(-VOCAB // V_BLK)


def _gather_sc(ids_flat, emb_table):
    info = plsc.get_sparse_core_info()
    nc, ns = info.num_cores, info.num_subcores
    nw = nc * ns
    b_per_w = N_TOK // nw
    mesh = plsc.VectorSubcoreMesh(core_axis_name="c", subcore_axis_name="s")

    @functools.partial(
        pl.kernel,
        mesh=mesh,
        out_type=jax.ShapeDtypeStruct((N_TOK, DIM), jnp.float32),
        scratch_types=[
            pltpu.VMEM((b_per_w,), jnp.int32),
            pltpu.VMEM((b_per_w, DIM), jnp.float32),
            pltpu.SemaphoreType.DMA,
        ],
    )
    def gather_kernel(ids_hbm, table_hbm, out_hbm, idx_v, rows_v, sem):
        wid = lax.axis_index("s") * nc + lax.axis_index("c")
        base = wid * b_per_w
        pltpu.sync_copy(ids_hbm.at[pl.ds(base, b_per_w)], idx_v)
        pltpu.async_copy(table_hbm.at[idx_v], rows_v, sem).wait()
        pltpu.sync_copy(rows_v, out_hbm.at[pl.ds(base, b_per_w)])

    return gather_kernel(ids_flat, emb_table)


V_PAD = N_VBLK * V_BLK


def _mm_body(x_ref, tbl_hbm, out_ref, tbl_vmem, sem):
    i = pl.program_id(0)

    @pl.when(i == 0)
    def _load_table():
        pltpu.make_async_copy(
            tbl_hbm, tbl_vmem.at[pl.ds(0, VOCAB), :], sem
        ).start()
        pltpu.make_async_copy(
            tbl_hbm, tbl_vmem.at[pl.ds(0, VOCAB), :], sem
        ).wait()

    out_ref[...] = lax.dot_general(
        x_ref[...].astype(jnp.bfloat16),
        tbl_vmem[pl.ds(i * V_BLK, V_BLK), :].astype(jnp.bfloat16),
        dimension_numbers=(((1,), (1,)), ((), ())),
        preferred_element_type=jnp.float32,
    )


def _matmul(x, emb_table):
    return pl.pallas_call(
        _mm_body,
        grid=(N_VBLK,),
        in_specs=[
            pl.BlockSpec((N_TOK, DIM), lambda i: (0, 0)),
            pl.BlockSpec(memory_space=pltpu.MemorySpace.HBM),
        ],
        out_specs=pl.BlockSpec((N_TOK, V_BLK), lambda i: (0, i)),
        out_shape=jax.ShapeDtypeStruct((N_TOK, VOCAB), jnp.float32),
        scratch_shapes=[
            pltpu.VMEM((V_PAD, DIM), jnp.float32),
            pltpu.SemaphoreType.DMA,
        ],
        compiler_params=pltpu.CompilerParams(vmem_limit_bytes=100 * 1024 * 1024),
    )(x, emb_table)


def kernel(ids, emb_table):
    ids_flat = ids.reshape(N_TOK).astype(jnp.int32)
    x = _gather_sc(ids_flat, emb_table)
    logits = _matmul(x, emb_table)
    return logits.reshape(B, L, VOCAB)


# speedup vs baseline: 1.0977x; 1.0977x over previous
import functools

import jax
import jax.numpy as jnp
from jax import lax
from jax.experimental import pallas as pl
from jax.experimental.pallas import tpu as pltpu
from jax.experimental.pallas import tpu_sc as plsc

VOCAB = 100000
DIM = 128
B, L = 64, 16
N_TOK = B * L

V_BLK = 6272
N_VBLK = -(-VOCAB // V_BLK)


def _gather_sc(ids_flat, emb_table):
    info = plsc.get_sparse_core_info()
    nc, ns = info.num_cores, info.num_subcores
    nw = nc * ns
    b_per_w = N_TOK // nw
    mesh = plsc.VectorSubcoreMesh(core_axis_name="c", subcore_axis_name="s")

    hw = b_per_w // 2

    @functools.partial(
        pl.kernel,
        mesh=mesh,
        out_type=jax.ShapeDtypeStruct((N_TOK, DIM), jnp.float32),
        scratch_types=[
            pltpu.VMEM((hw,), jnp.int32),
            pltpu.VMEM((hw,), jnp.int32),
            pltpu.VMEM((hw, DIM), jnp.float32),
            pltpu.VMEM((hw, DIM), jnp.float32),
            pltpu.SemaphoreType.DMA,
            pltpu.SemaphoreType.DMA,
            pltpu.SemaphoreType.DMA,
            pltpu.SemaphoreType.DMA,
        ],
    )
    def gather_kernel(ids_hbm, table_hbm, out_hbm,
                      idx0, idx1, rows0, rows1, s0, s1, s2, s3):
        wid = lax.axis_index("s") * nc + lax.axis_index("c")
        base = wid * b_per_w
        ci0 = pltpu.async_copy(ids_hbm.at[pl.ds(base, hw)], idx0, s0)
        ci1 = pltpu.async_copy(ids_hbm.at[pl.ds(base + hw, hw)], idx1, s1)
        ci0.wait()
        g0 = pltpu.async_copy(table_hbm.at[idx0], rows0, s2)
        ci1.wait()
        g1 = pltpu.async_copy(table_hbm.at[idx1], rows1, s3)
        g0.wait()
        o0 = pltpu.async_copy(rows0, out_hbm.at[pl.ds(base, hw)], s0)
        g1.wait()
        o1 = pltpu.async_copy(rows1, out_hbm.at[pl.ds(base + hw, hw)], s1)
        o0.wait()
        o1.wait()

    return gather_kernel(ids_flat, emb_table)


def _mm_body(x_ref, tbl_ref, out_ref):
    out_ref[...] = lax.dot_general(
        x_ref[...].astype(jnp.bfloat16),
        tbl_ref[...].astype(jnp.bfloat16),
        dimension_numbers=(((1,), (1,)), ((), ())),
        preferred_element_type=jnp.float32,
    )


def _matmul(x, emb_table):
    return pl.pallas_call(
        _mm_body,
        grid=(N_VBLK,),
        in_specs=[
            pl.BlockSpec((N_TOK, DIM), lambda i: (0, 0)),
            pl.BlockSpec((V_BLK, DIM), lambda i: (i, 0)),
        ],
        out_specs=pl.BlockSpec((N_TOK, V_BLK), lambda i: (0, i)),
        out_shape=jax.ShapeDtypeStruct((N_TOK, VOCAB), jnp.float32),
        compiler_params=pltpu.CompilerParams(vmem_limit_bytes=100 * 1024 * 1024),
    )(x, emb_table)


def kernel(ids, emb_table):
    ids_flat = ids.reshape(N_TOK).astype(jnp.int32)
    x = _gather_sc(ids_flat, emb_table)
    logits = _matmul(x, emb_table)
    return logits.reshape(B, L, VOCAB)


# speedup vs baseline: 1.1024x; 1.0043x over previous
import functools

import jax
import jax.numpy as jnp
from jax import lax
from jax.experimental import pallas as pl
from jax.experimental.pallas import tpu as pltpu
from jax.experimental.pallas import tpu_sc as plsc

VOCAB = 100000
DIM = 128
B, L = 64, 16
N_TOK = B * L

V_BLK = 6272
N_VBLK = -(-VOCAB // V_BLK)


def _gather_sc(ids_flat, emb_table):
    info = plsc.get_sparse_core_info()
    nc, ns = info.num_cores, info.num_subcores
    nw = nc * ns
    b_per_w = N_TOK // nw
    mesh = plsc.VectorSubcoreMesh(core_axis_name="c", subcore_axis_name="s")

    @functools.partial(
        pl.kernel,
        mesh=mesh,
        out_type=jax.ShapeDtypeStruct((N_TOK, DIM), jnp.float32),
        scratch_types=[
            pltpu.VMEM((b_per_w,), jnp.int32),
            pltpu.VMEM((b_per_w, DIM), jnp.float32),
            pltpu.SemaphoreType.DMA,
        ],
    )
    def gather_kernel(ids_hbm, table_hbm, out_hbm, idx_v, rows_v, sem):
        wid = lax.axis_index("s") * nc + lax.axis_index("c")
        base = wid * b_per_w
        pltpu.sync_copy(ids_hbm.at[pl.ds(base, b_per_w)], idx_v)
        pltpu.async_copy(table_hbm.at[idx_v], rows_v, sem).wait()
        pltpu.sync_copy(rows_v, out_hbm.at[pl.ds(base, b_per_w)])

    return gather_kernel(ids_flat, emb_table)


def _mm_body(x_ref, tbl_ref, out_ref):
    out_ref[...] = lax.dot_general(
        x_ref[...].astype(jnp.bfloat16),
        tbl_ref[...].astype(jnp.bfloat16),
        dimension_numbers=(((1,), (1,)), ((), ())),
        preferred_element_type=jnp.float32,
    )


def _matmul(x, emb_table):
    return pl.pallas_call(
        _mm_body,
        grid=(N_VBLK,),
        in_specs=[
            pl.BlockSpec((N_TOK, DIM), lambda i: (0, 0)),
            pl.BlockSpec((V_BLK, DIM), lambda i: (i, 0)),
        ],
        out_specs=pl.BlockSpec((N_TOK, V_BLK), lambda i: (0, i)),
        out_shape=jax.ShapeDtypeStruct((N_TOK, VOCAB), jnp.float32),
        compiler_params=pltpu.CompilerParams(vmem_limit_bytes=100 * 1024 * 1024),
    )(x, emb_table)


def kernel(ids, emb_table):
    ids_flat = ids.reshape(N_TOK).astype(jnp.int32)
    x = _gather_sc(ids_flat, emb_table)
    logits = _matmul(x, emb_table)
    return logits.reshape(B, L, VOCAB)


# speedup vs baseline: 1.1072x; 1.0043x over previous
import functools

import jax
import jax.numpy as jnp
from jax import lax
from jax.experimental import pallas as pl
from jax.experimental.pallas import tpu as pltpu
from jax.experimental.pallas import tpu_sc as plsc

VOCAB = 100000
DIM = 128
B, L = 64, 16
N_TOK = B * L

V_BLK = 7168
N_VBLK = -(-VOCAB // V_BLK)


def _gather_sc(ids_flat, emb_table):
    info = plsc.get_sparse_core_info()
    nc, ns = info.num_cores, info.num_subcores
    nw = nc * ns
    b_per_w = N_TOK // nw
    mesh = plsc.VectorSubcoreMesh(core_axis_name="c", subcore_axis_name="s")

    @functools.partial(
        pl.kernel,
        mesh=mesh,
        out_type=jax.ShapeDtypeStruct((N_TOK, DIM), jnp.float32),
        scratch_types=[
            pltpu.VMEM((b_per_w,), jnp.int32),
            pltpu.VMEM((b_per_w, DIM), jnp.float32),
            pltpu.SemaphoreType.DMA,
        ],
    )
    def gather_kernel(ids_hbm, table_hbm, out_hbm, idx_v, rows_v, sem):
        wid = lax.axis_index("s") * nc + lax.axis_index("c")
        base = wid * b_per_w
        pltpu.sync_copy(ids_hbm.at[pl.ds(base, b_per_w)], idx_v)
        pltpu.async_copy(table_hbm.at[idx_v], rows_v, sem).wait()
        pltpu.sync_copy(rows_v, out_hbm.at[pl.ds(base, b_per_w)])

    return gather_kernel(ids_flat, emb_table)


def _mm_body(x_ref, tbl_ref, out_ref):
    out_ref[...] = lax.dot_general(
        x_ref[...].astype(jnp.bfloat16),
        tbl_ref[...].astype(jnp.bfloat16),
        dimension_numbers=(((1,), (1,)), ((), ())),
        preferred_element_type=jnp.float32,
    )


def _matmul(x, emb_table):
    return pl.pallas_call(
        _mm_body,
        grid=(N_VBLK,),
        in_specs=[
            pl.BlockSpec((N_TOK, DIM), lambda i: (0, 0)),
            pl.BlockSpec((V_BLK, DIM), lambda i: (i, 0)),
        ],
        out_specs=pl.BlockSpec((N_TOK, V_BLK), lambda i: (0, i)),
        out_shape=jax.ShapeDtypeStruct((N_TOK, VOCAB), jnp.float32),
        compiler_params=pltpu.CompilerParams(vmem_limit_bytes=100 * 1024 * 1024),
    )(x, emb_table)


def kernel(ids, emb_table):
    ids_flat = ids.reshape(N_TOK).astype(jnp.int32)
    x = _gather_sc(ids_flat, emb_table)
    logits = _matmul(x, emb_table)
    return logits.reshape(B, L, VOCAB)


# speedup vs baseline: 1.1207x; 1.0122x over previous
import functools

import jax
import jax.numpy as jnp
from jax import lax
from jax.experimental import pallas as pl
from jax.experimental.pallas import tpu as pltpu
from jax.experimental.pallas import tpu_sc as plsc

VOCAB = 100000
DIM = 128
B, L = 64, 16
N_TOK = B * L

V_BLK = 6272
N_VBLK = -(-VOCAB // V_BLK)


def _gather_sc(ids_flat, emb_table):
    info = plsc.get_sparse_core_info()
    nc, ns = info.num_cores, info.num_subcores
    nw = nc * ns
    b_per_w = N_TOK // nw
    mesh = plsc.VectorSubcoreMesh(core_axis_name="c", subcore_axis_name="s")

    @functools.partial(
        pl.kernel,
        mesh=mesh,
        out_type=jax.ShapeDtypeStruct((N_TOK, DIM), jnp.float32),
        scratch_types=[
            pltpu.VMEM((b_per_w,), jnp.int32),
            pltpu.VMEM((b_per_w, DIM), jnp.float32),
            pltpu.SemaphoreType.DMA,
        ],
    )
    def gather_kernel(ids_hbm, table_hbm, out_hbm, idx_v, rows_v, sem):
        wid = lax.axis_index("s") * nc + lax.axis_index("c")
        base = wid * b_per_w
        pltpu.sync_copy(ids_hbm.at[pl.ds(base, b_per_w)], idx_v)
        pltpu.async_copy(table_hbm.at[idx_v], rows_v, sem).wait()
        pltpu.sync_copy(rows_v, out_hbm.at[pl.ds(base, b_per_w)])

    return gather_kernel(ids_flat, emb_table)


def _mm_body(x_ref, tbl_ref, out_ref):
    out_ref[...] = lax.dot_general(
        x_ref[...].astype(jnp.bfloat16),
        tbl_ref[...].astype(jnp.bfloat16),
        dimension_numbers=(((1,), (1,)), ((), ())),
        preferred_element_type=jnp.float32,
    )


def _matmul(x, emb_table):
    return pl.pallas_call(
        _mm_body,
        grid=(N_VBLK,),
        in_specs=[
            pl.BlockSpec((N_TOK, DIM), lambda i: (0, 0)),
            pl.BlockSpec((V_BLK, DIM), lambda i: (i, 0)),
        ],
        out_specs=pl.BlockSpec((N_TOK, V_BLK), lambda i: (0, i)),
        out_shape=jax.ShapeDtypeStruct((N_TOK, VOCAB), jnp.float32),
        compiler_params=pltpu.CompilerParams(vmem_limit_bytes=100 * 1024 * 1024),
    )(x, emb_table)


def kernel(ids, emb_table):
    ids_flat = ids.reshape(N_TOK).astype(jnp.int32)
    x = _gather_sc(ids_flat, emb_table)
    logits = _matmul(x, emb_table)
    return logits.reshape(B, L, VOCAB)
